# baseline (speedup 1.0000x reference)
import functools

import jax
import jax.numpy as jnp
from jax import lax
from jax.experimental import pallas as pl
from jax.experimental.pallas import tpu as pltpu
from jax.experimental.pallas import tpu_sc as plsc

B = 4096
L = 200
D = 64
NUM_WORKERS = 32
RPW = B // NUM_WORKERS
CA = 104
CB = L - CA
LANES = 16
NCHUNK = D // LANES
V = 1000000
NPAIR = V // 2
TBLK = 128
NFULL = V // TBLK
PAIRS_PER_BLK = TBLK // 2


def _worker_id():
    return lax.axis_index("s") * 2 + lax.axis_index("c")


def _transpose_body(tab3_hbm, tail_hbm, pairs_hbm,
                    blk, outblk, tailv, insems, outsems):
    wid = _worker_id()
    lane = lax.broadcasted_iota(jnp.int32, (LANES,), 0)
    zero = lane * 0
    in0, in1 = insems
    out0, out1 = outsems

    svec = [(c * LANES + jnp.arange(LANES, dtype=jnp.int32)) // 8
            for c in range(NCHUNK)]
    dvec = [(c * LANES + jnp.arange(LANES, dtype=jnp.int32)) % 8
            for c in range(NCHUNK)]

    nblk = (NFULL - wid + NUM_WORKERS - 1) // NUM_WORKERS
    kmax = wid + (nblk - 1) * NUM_WORKERS

    def kof(i):
        return jnp.minimum(wid + i * NUM_WORKERS, kmax)

    def issue_in(k, slot, sem):
        pltpu.async_copy(tab3_hbm.at[:, :, pl.ds(k * TBLK, TBLK)],
                         blk.at[slot], sem)

    def drain_in(slot, sem):
        pltpu.make_async_copy(tab3_hbm.at[:, :, pl.ds(0, TBLK)],
                              blk.at[slot], sem).wait()

    def issue_out(k, slot, sem):
        pltpu.async_copy(outblk.at[slot],
                         pairs_hbm.at[pl.ds(k * PAIRS_PER_BLK,
                                            PAIRS_PER_BLK)], sem)

    def drain_out(slot, sem):
        pltpu.make_async_copy(outblk.at[slot],
                              pairs_hbm.at[pl.ds(0, PAIRS_PER_BLK)],
                              sem).wait()

    def compute(slot, oslot):
        slot_b = zero + slot

        def pair(j, carry):
            for h in range(2):
                t_b = zero + 2 * j + h
                for c in range(NCHUNK):
                    vals = plsc.load_gather(
                        blk, [slot_b, svec[c], dvec[c], t_b])
                    outblk[oslot, j, pl.ds(h * D + c * LANES, LANES)] = vals
            return carry

        lax.fori_loop(0, PAIRS_PER_BLK, pair, 0, unroll=2)

    issue_in(kof(0), 0, in0)
    npair_iter = (nblk + 1) // 2

    def loop(p, carry):
        i0 = 2 * p
        issue_in(kof(i0 + 1), 1, in1)
        drain_in(0, in0)

        @pl.when(p > 0)
        def _():
            drain_out(0, out0)

        compute(0, 0)
        issue_out(kof(i0), 0, out0)

        issue_in(kof(i0 + 2), 0, in0)
        drain_in(1, in1)

        @pl.when(p > 0)
        def _():
            drain_out(1, out1)

        compute(1, 1)
        issue_out(kof(i0 + 1), 1, out1)
        return carry

    lax.fori_loop(0, npair_iter, loop, 0)
    drain_in(0, in0)
    drain_out(0, out0)
    drain_out(1, out1)

    @pl.when(wid == 0)
    def _():
        pltpu.sync_copy(tail_hbm, tailv)
        pltpu.sync_copy(tailv, pairs_hbm.at[pl.ds(NFULL * PAIRS_PER_BLK,
                                                  V % TBLK // 2)])


def _gather_body(rowsa_hbm, rowsb_hbm, coff_hbm, len_hbm, table_hbm, out_hbm,
                 idxa_v, idxb_v, coff_v, len_v, inv_v, bufa, bufb, outw,
                 sems):
    wid = _worker_id()

    pltpu.sync_copy(rowsa_hbm.at[wid], idxa_v)
    pltpu.sync_copy(rowsb_hbm.at[wid], idxb_v)
    pltpu.sync_copy(coff_hbm.at[wid], coff_v)
    pltpu.sync_copy(len_hbm.at[wid], len_v)

    for g in range(RPW // LANES):
        lens16 = len_v[pl.ds(g * LANES, LANES)]
        inv_v[pl.ds(g * LANES, LANES)] = (
            1.0 / jnp.maximum(lens16, 1).astype(jnp.float32))

    lane = lax.broadcasted_iota(jnp.int32, (LANES,), 0)
    zero = lane * 0
    sem0, sem1 = sems

    def issue(r, slot, sem):
        pltpu.async_copy(table_hbm.at[idxa_v.at[r]], bufa.at[slot], sem)
        pltpu.async_copy(table_hbm.at[idxb_v.at[r]], bufb.at[slot], sem)

    def drain(slot, sem):
        pltpu.make_async_copy(table_hbm.at[pl.ds(0, CA)],
                              bufa.at[slot], sem).wait()
        pltpu.make_async_copy(table_hbm.at[pl.ds(0, CB)],
                              bufb.at[slot], sem).wait()

    def accumulate(r, slot):
        r_b = zero + r
        slot_b = zero + slot

        def make_acc(buf, base):
            def acc_body(t, accs):
                coff = plsc.load_gather(coff_v, [zero + base + t, r_b])
                new = []
                for c in range(NCHUNK):
                    col = coff + (c * LANES) + lane
                    new.append(accs[c] + plsc.load_gather(
                        buf, [slot_b, zero + t, col]))
                return tuple(new)
            return acc_body

        accs = tuple(jnp.zeros((LANES,), jnp.float32) for _ in range(NCHUNK))
        accs = lax.fori_loop(0, CA, make_acc(bufa, 0), accs, unroll=4)
        accs = lax.fori_loop(0, CB, make_acc(bufb, CA), accs, unroll=4)

        sinv = plsc.load_gather(inv_v, [r_b])
        for c in range(NCHUNK):
            outw[r, pl.ds(c * LANES, LANES)] = accs[c] * sinv

    issue(0, 0, sem0)

    def pair_body(p, carry):
        r0 = 2 * p
        r1 = r0 + 1
        issue(r1, 1, sem1)
        drain(0, sem0)
        accumulate(r0, 0)
        issue(jnp.minimum(r1 + 1, RPW - 1), 0, sem0)
        drain(1, sem1)
        accumulate(r1, 1)
        return carry

    lax.fori_loop(0, RPW // 2, pair_body, 0)
    drain(0, sem0)
    pltpu.sync_copy(outw, out_hbm.at[pl.ds(wid * RPW, RPW)])


@functools.partial(jax.jit, static_argnames=("interpret",))
def _run(tokens, token_lengths, table, interpret=False):
    mesh = plsc.VectorSubcoreMesh(core_axis_name="c", subcore_axis_name="s",
                                  num_cores=2, num_subcores=16)
    params = pltpu.CompilerParams(needs_layout_passes=False,
                                  use_tc_tiling_on_sc=True)

    tab3 = table.T.reshape(D // 8, 8, V)
    tail = table[NFULL * TBLK:].reshape(V % TBLK // 2, 2 * D)

    transpose_k = pl.kernel(
        _transpose_body,
        out_type=jax.ShapeDtypeStruct((NPAIR, 2 * D), jnp.float32),
        mesh=mesh,
        compiler_params=params,
        scratch_types=[
            pltpu.VMEM((2, D // 8, 8, TBLK), jnp.float32),
            pltpu.VMEM((2, PAIRS_PER_BLK, 2 * D), jnp.float32),
            pltpu.VMEM((V % TBLK // 2, 2 * D), jnp.float32),
            (pltpu.SemaphoreType.DMA, pltpu.SemaphoreType.DMA),
            (pltpu.SemaphoreType.DMA, pltpu.SemaphoreType.DMA),
        ],
        interpret=interpret,
    )
    pairs_tab = transpose_k(tab3, tail)

    rowsa = tokens[:, :CA].reshape(NUM_WORKERS, RPW, CA) >> 1
    rowsb = tokens[:, CA:].reshape(NUM_WORKERS, RPW, CB) >> 1
    coff = ((tokens & 1) * D).reshape(NUM_WORKERS, RPW, L)
    coff = coff.transpose(0, 2, 1)
    lens = token_lengths.reshape(NUM_WORKERS, RPW)
    gather_k = pl.kernel(
        _gather_body,
        out_type=jax.ShapeDtypeStruct((B, D), jnp.float32),
        mesh=mesh,
        compiler_params=params,
        scratch_types=[
            pltpu.VMEM((RPW, CA), jnp.int32),
            pltpu.VMEM((RPW, CB), jnp.int32),
            pltpu.VMEM((L, RPW), jnp.int32),
            pltpu.VMEM((RPW,), jnp.int32),
            pltpu.VMEM((RPW,), jnp.float32),
            pltpu.VMEM((2, CA, 2 * D), jnp.float32),
            pltpu.VMEM((2, CB, 2 * D), jnp.float32),
            pltpu.VMEM((RPW, D), jnp.float32),
            (pltpu.SemaphoreType.DMA, pltpu.SemaphoreType.DMA),
        ],
        interpret=interpret,
    )
    return gather_k(rowsa, rowsb, coff, lens, pairs_tab)


def kernel(tokens, token_lengths, table):
    return _run(tokens, token_lengths, table)


# speedup vs baseline: 1.1931x; 1.1931x over previous
import functools

import jax
import jax.numpy as jnp
from jax import lax
from jax.experimental import pallas as pl
from jax.experimental.pallas import tpu as pltpu
from jax.experimental.pallas import tpu_sc as plsc

B = 4096
L = 200
D = 64
NUM_WORKERS = 32
RPW = B // NUM_WORKERS
CA = 104
CB = L - CA
LANES = 16
NCHUNK = D // LANES
V = 1000000
NPAIR = V // 2
TBLK = 128
NFULL = V // TBLK
PAIRS_PER_BLK = TBLK // 2


def _worker_id():
    return lax.axis_index("s") * 2 + lax.axis_index("c")


def _transpose_body(tab3_hbm, tail_hbm, pairs_hbm,
                    blk, outblk, tailv, insems, outsems):
    wid = _worker_id()
    lane = lax.broadcasted_iota(jnp.int32, (LANES,), 0)
    zero = lane * 0
    in0, in1 = insems
    out0, out1 = outsems

    parity = lane & 1
    halflane = lane >> 1

    nblk = (NFULL - wid + NUM_WORKERS - 1) // NUM_WORKERS
    kmax = wid + (nblk - 1) * NUM_WORKERS

    def kof(i):
        return jnp.minimum(wid + i * NUM_WORKERS, kmax)

    def issue_in(k, slot, sem):
        pltpu.async_copy(tab3_hbm.at[:, :, pl.ds(k * TBLK, TBLK)],
                         blk.at[slot], sem)

    def drain_in(slot, sem):
        pltpu.make_async_copy(tab3_hbm.at[:, :, pl.ds(0, TBLK)],
                              blk.at[slot], sem).wait()

    def issue_out(k, slot, sem):
        pltpu.async_copy(outblk.at[slot],
                         pairs_hbm.at[pl.ds(k * PAIRS_PER_BLK,
                                            PAIRS_PER_BLK)], sem)

    def drain_out(slot, sem):
        pltpu.make_async_copy(outblk.at[slot],
                              pairs_hbm.at[pl.ds(0, PAIRS_PER_BLK)],
                              sem).wait()

    def compute(slot, oslot):
        oslot_b = zero + oslot
        for d in range(D):
            colv = parity * D + d
            for g in range(TBLK // LANES):
                vals = blk[slot, d // 8, d % 8, pl.ds(g * LANES, LANES)]
                rowv = halflane + (g * LANES // 2)
                plsc.store_scatter(outblk, [oslot_b, rowv, colv], vals)

    issue_in(kof(0), 0, in0)
    npair_iter = (nblk + 1) // 2

    def loop(p, carry):
        i0 = 2 * p
        issue_in(kof(i0 + 1), 1, in1)
        drain_in(0, in0)

        @pl.when(p > 0)
        def _():
            drain_out(0, out0)

        compute(0, 0)
        issue_out(kof(i0), 0, out0)

        issue_in(kof(i0 + 2), 0, in0)
        drain_in(1, in1)

        @pl.when(p > 0)
        def _():
            drain_out(1, out1)

        compute(1, 1)
        issue_out(kof(i0 + 1), 1, out1)
        return carry

    lax.fori_loop(0, npair_iter, loop, 0)
    drain_in(0, in0)
    drain_out(0, out0)
    drain_out(1, out1)

    @pl.when(wid == 0)
    def _():
        pltpu.sync_copy(tail_hbm, tailv)
        pltpu.sync_copy(tailv, pairs_hbm.at[pl.ds(NFULL * PAIRS_PER_BLK,
                                                  V % TBLK // 2)])


def _gather_body(rowsa_hbm, rowsb_hbm, coff_hbm, len_hbm, table_hbm, out_hbm,
                 idxa_v, idxb_v, coff_v, len_v, inv_v, bufa, bufb, outw,
                 sems):
    wid = _worker_id()

    pltpu.sync_copy(rowsa_hbm.at[wid], idxa_v)
    pltpu.sync_copy(rowsb_hbm.at[wid], idxb_v)
    pltpu.sync_copy(coff_hbm.at[wid], coff_v)
    pltpu.sync_copy(len_hbm.at[wid], len_v)

    for g in range(RPW // LANES):
        lens16 = len_v[pl.ds(g * LANES, LANES)]
        inv_v[pl.ds(g * LANES, LANES)] = (
            1.0 / jnp.maximum(lens16, 1).astype(jnp.float32))

    lane = lax.broadcasted_iota(jnp.int32, (LANES,), 0)
    zero = lane * 0
    sem0, sem1 = sems

    def issue(r, slot, sem):
        pltpu.async_copy(table_hbm.at[idxa_v.at[r]], bufa.at[slot], sem)
        pltpu.async_copy(table_hbm.at[idxb_v.at[r]], bufb.at[slot], sem)

    def drain(slot, sem):
        pltpu.make_async_copy(table_hbm.at[pl.ds(0, CA)],
                              bufa.at[slot], sem).wait()
        pltpu.make_async_copy(table_hbm.at[pl.ds(0, CB)],
                              bufb.at[slot], sem).wait()

    def accumulate(r, slot):
        r_b = zero + r
        slot_b = zero + slot

        def make_acc(buf, base):
            def acc_body(t, accs):
                coff = plsc.load_gather(coff_v, [zero + base + t, r_b])
                new = []
                for c in range(NCHUNK):
                    col = coff + (c * LANES) + lane
                    new.append(accs[c] + plsc.load_gather(
                        buf, [slot_b, zero + t, col]))
                return tuple(new)
            return acc_body

        accs = tuple(jnp.zeros((LANES,), jnp.float32) for _ in range(NCHUNK))
        accs = lax.fori_loop(0, CA, make_acc(bufa, 0), accs, unroll=4)
        accs = lax.fori_loop(0, CB, make_acc(bufb, CA), accs, unroll=4)

        sinv = plsc.load_gather(inv_v, [r_b])
        for c in range(NCHUNK):
            outw[r, pl.ds(c * LANES, LANES)] = accs[c] * sinv

    issue(0, 0, sem0)

    def pair_body(p, carry):
        r0 = 2 * p
        r1 = r0 + 1
        issue(r1, 1, sem1)
        drain(0, sem0)
        accumulate(r0, 0)
        issue(jnp.minimum(r1 + 1, RPW - 1), 0, sem0)
        drain(1, sem1)
        accumulate(r1, 1)
        return carry

    lax.fori_loop(0, RPW // 2, pair_body, 0)
    drain(0, sem0)
    pltpu.sync_copy(outw, out_hbm.at[pl.ds(wid * RPW, RPW)])


@functools.partial(jax.jit, static_argnames=("interpret",))
def _run(tokens, token_lengths, table, interpret=False):
    mesh = plsc.VectorSubcoreMesh(core_axis_name="c", subcore_axis_name="s",
                                  num_cores=2, num_subcores=16)
    params = pltpu.CompilerParams(needs_layout_passes=False,
                                  use_tc_tiling_on_sc=True)

    tab3 = table.T.reshape(D // 8, 8, V)
    tail = table[NFULL * TBLK:].reshape(V % TBLK // 2, 2 * D)

    transpose_k = pl.kernel(
        _transpose_body,
        out_type=jax.ShapeDtypeStruct((NPAIR, 2 * D), jnp.float32),
        mesh=mesh,
        compiler_params=params,
        scratch_types=[
            pltpu.VMEM((2, D // 8, 8, TBLK), jnp.float32),
            pltpu.VMEM((2, PAIRS_PER_BLK, 2 * D), jnp.float32),
            pltpu.VMEM((V % TBLK // 2, 2 * D), jnp.float32),
            (pltpu.SemaphoreType.DMA, pltpu.SemaphoreType.DMA),
            (pltpu.SemaphoreType.DMA, pltpu.SemaphoreType.DMA),
        ],
        interpret=interpret,
    )
    pairs_tab = transpose_k(tab3, tail)

    rowsa = tokens[:, :CA].reshape(NUM_WORKERS, RPW, CA) >> 1
    rowsb = tokens[:, CA:].reshape(NUM_WORKERS, RPW, CB) >> 1
    coff = ((tokens & 1) * D).reshape(NUM_WORKERS, RPW, L)
    coff = coff.transpose(0, 2, 1)
    lens = token_lengths.reshape(NUM_WORKERS, RPW)
    gather_k = pl.kernel(
        _gather_body,
        out_type=jax.ShapeDtypeStruct((B, D), jnp.float32),
        mesh=mesh,
        compiler_params=params,
        scratch_types=[
            pltpu.VMEM((RPW, CA), jnp.int32),
            pltpu.VMEM((RPW, CB), jnp.int32),
            pltpu.VMEM((L, RPW), jnp.int32),
            pltpu.VMEM((RPW,), jnp.int32),
            pltpu.VMEM((RPW,), jnp.float32),
            pltpu.VMEM((2, CA, 2 * D), jnp.float32),
            pltpu.VMEM((2, CB, 2 * D), jnp.float32),
            pltpu.VMEM((RPW, D), jnp.float32),
            (pltpu.SemaphoreType.DMA, pltpu.SemaphoreType.DMA),
        ],
        interpret=interpret,
    )
    return gather_k(rowsa, rowsb, coff, lens, pairs_tab)


def kernel(tokens, token_lengths, table):
    return _run(tokens, token_lengths, table)


# speedup vs baseline: 3.1247x; 2.6190x over previous
import functools

import jax
import jax.numpy as jnp
from jax import lax
from jax.experimental import pallas as pl
from jax.experimental.pallas import tpu as pltpu
from jax.experimental.pallas import tpu_sc as plsc

B = 4096
L = 200
D = 64
NUM_WORKERS = 32
RPW = B // NUM_WORKERS
CA = 104
CB = L - CA
LANES = 16
NCHUNK = D // LANES
V = 1000000
TW = 2048
NBLK = V // TW
NSTEP = NBLK // 2 + 1
VFULL = NBLK * TW
TAILN = (V - VFULL) // 2
NPAIR = NSTEP * TW


def _worker_id():
    return lax.axis_index("s") * 2 + lax.axis_index("c")


def _tr_body(srcl_ref, srcr_ref, tall_ref, talr_ref, dst_ref):
    i = pl.program_id(0)

    @pl.when(i < NSTEP - 1)
    def _():
        dst_ref[...] = jnp.concatenate(
            [jnp.swapaxes(srcl_ref[...], 0, 1),
             jnp.swapaxes(srcr_ref[...], 0, 1)], axis=1)

    @pl.when(i == NSTEP - 1)
    def _():
        dst_ref[pl.ds(0, TAILN), :] = jnp.concatenate(
            [jnp.swapaxes(tall_ref[...], 0, 1),
             jnp.swapaxes(talr_ref[...], 0, 1)], axis=1)


def _gather_body(rowsa_hbm, rowsb_hbm, coff_hbm, len_hbm, table_hbm, out_hbm,
                 idxa_v, idxb_v, coff_v, len_v, inv_v, bufa, bufb, outw,
                 sems):
    wid = _worker_id()

    pltpu.sync_copy(rowsa_hbm.at[wid], idxa_v)
    pltpu.sync_copy(rowsb_hbm.at[wid], idxb_v)
    pltpu.sync_copy(coff_hbm.at[wid], coff_v)
    pltpu.sync_copy(len_hbm.at[wid], len_v)

    for g in range(RPW // LANES):
        lens16 = len_v[pl.ds(g * LANES, LANES)]
        inv_v[pl.ds(g * LANES, LANES)] = (
            1.0 / jnp.maximum(lens16, 1).astype(jnp.float32))

    lane = lax.broadcasted_iota(jnp.int32, (LANES,), 0)
    zero = lane * 0
    sem0, sem1 = sems

    def issue(r, slot, sem):
        pltpu.async_copy(table_hbm.at[idxa_v.at[r]], bufa.at[slot], sem)
        pltpu.async_copy(table_hbm.at[idxb_v.at[r]], bufb.at[slot], sem)

    def drain(slot, sem):
        pltpu.make_async_copy(table_hbm.at[pl.ds(0, CA)],
                              bufa.at[slot], sem).wait()
        pltpu.make_async_copy(table_hbm.at[pl.ds(0, CB)],
                              bufb.at[slot], sem).wait()

    def accumulate(r, slot):
        r_b = zero + r
        slot_b = zero + slot

        def make_acc(buf, base):
            def acc_body(t, accs):
                coff = plsc.load_gather(coff_v, [zero + base + t, r_b])
                new = []
                for c in range(NCHUNK):
                    col = coff + (c * LANES) + lane
                    new.append(accs[c] + plsc.load_gather(
                        buf, [slot_b, zero + t, col]))
                return tuple(new)
            return acc_body

        accs = tuple(jnp.zeros((LANES,), jnp.float32) for _ in range(NCHUNK))
        accs = lax.fori_loop(0, CA, make_acc(bufa, 0), accs, unroll=4)
        accs = lax.fori_loop(0, CB, make_acc(bufb, CA), accs, unroll=4)

        sinv = plsc.load_gather(inv_v, [r_b])
        for c in range(NCHUNK):
            outw[r, pl.ds(c * LANES, LANES)] = accs[c] * sinv

    issue(0, 0, sem0)

    def pair_body(p, carry):
        r0 = 2 * p
        r1 = r0 + 1
        issue(r1, 1, sem1)
        drain(0, sem0)
        accumulate(r0, 0)
        issue(jnp.minimum(r1 + 1, RPW - 1), 0, sem0)
        drain(1, sem1)
        accumulate(r1, 1)
        return carry

    lax.fori_loop(0, RPW // 2, pair_body, 0)
    drain(0, sem0)
    pltpu.sync_copy(outw, out_hbm.at[pl.ds(wid * RPW, RPW)])


@functools.partial(jax.jit, static_argnames=("interpret",))
def _run(tokens, token_lengths, table, interpret=False):
    mesh = plsc.VectorSubcoreMesh(core_axis_name="c", subcore_axis_name="s",
                                  num_cores=2, num_subcores=16)
    params = pltpu.CompilerParams(needs_layout_passes=False,
                                  use_tc_tiling_on_sc=True)

    tabt = table.T
    tall = tabt[:, VFULL:VFULL + TAILN]
    talr = tabt[:, VFULL + TAILN:]
    pairs_tab = pl.pallas_call(
        _tr_body,
        grid=(NSTEP,),
        in_specs=[
            pl.BlockSpec((D, TW), lambda i: (0, jnp.minimum(2 * i,
                                                            NBLK - 2))),
            pl.BlockSpec((D, TW), lambda i: (0, jnp.minimum(2 * i + 1,
                                                            NBLK - 1))),
            pl.BlockSpec((D, TAILN), lambda i: (0, 0)),
            pl.BlockSpec((D, TAILN), lambda i: (0, 0)),
        ],
        out_specs=pl.BlockSpec((TW, 2 * D), lambda i: (i, 0)),
        out_shape=jax.ShapeDtypeStruct((NPAIR, 2 * D), jnp.float32),
        interpret=interpret,
    )(tabt, tabt, tall, talr)

    blk = tokens >> 11
    half = blk & 1
    prow = (blk >> 1) * TW + (tokens & (TW - 1))
    tail_j = tokens - VFULL
    is_tail = tokens >= VFULL
    half = jnp.where(is_tail, tail_j // TAILN, half)
    prow = jnp.where(is_tail, (NSTEP - 1) * TW + tail_j % TAILN, prow)
    rowsa = prow[:, :CA].reshape(NUM_WORKERS, RPW, CA)
    rowsb = prow[:, CA:].reshape(NUM_WORKERS, RPW, CB)
    coff = (half * D).reshape(NUM_WORKERS, RPW, L)
    coff = coff.transpose(0, 2, 1)
    lens = token_lengths.reshape(NUM_WORKERS, RPW)
    gather_k = pl.kernel(
        _gather_body,
        out_type=jax.ShapeDtypeStruct((B, D), jnp.float32),
        mesh=mesh,
        compiler_params=params,
        scratch_types=[
            pltpu.VMEM((RPW, CA), jnp.int32),
            pltpu.VMEM((RPW, CB), jnp.int32),
            pltpu.VMEM((L, RPW), jnp.int32),
            pltpu.VMEM((RPW,), jnp.int32),
            pltpu.VMEM((RPW,), jnp.float32),
            pltpu.VMEM((2, CA, 2 * D), jnp.float32),
            pltpu.VMEM((2, CB, 2 * D), jnp.float32),
            pltpu.VMEM((RPW, D), jnp.float32),
            (pltpu.SemaphoreType.DMA, pltpu.SemaphoreType.DMA),
        ],
        interpret=interpret,
    )
    return gather_k(rowsa, rowsb, coff, lens, pairs_tab)


def kernel(tokens, token_lengths, table):
    return _run(tokens, token_lengths, table)


# speedup vs baseline: 3.6476x; 1.1674x over previous
import functools

import jax
import jax.numpy as jnp
from jax import lax
from jax.experimental import pallas as pl
from jax.experimental.pallas import tpu as pltpu
from jax.experimental.pallas import tpu_sc as plsc

B = 4096
L = 200
D = 64
NUM_WORKERS = 32
RPW = B // NUM_WORKERS
CA = 104
CB = L - CA
LANES = 16
NCHUNK = D // LANES
V = 1000000
TW = 2048
NBLK = V // TW
NSTEP = NBLK // 2 + 1
VFULL = NBLK * TW
TAILN = (V - VFULL) // 2
NPAIR = NSTEP * TW


def _worker_id():
    return lax.axis_index("s") * 2 + lax.axis_index("c")


def _tr_body(srcl_ref, srcr_ref, tall_ref, talr_ref, dst_ref):
    i = pl.program_id(0)

    @pl.when(i < NSTEP - 1)
    def _():
        dst_ref[...] = jnp.concatenate(
            [jnp.swapaxes(srcl_ref[...], 0, 1),
             jnp.swapaxes(srcr_ref[...], 0, 1)], axis=1)

    @pl.when(i == NSTEP - 1)
    def _():
        dst_ref[pl.ds(0, TAILN), :] = jnp.concatenate(
            [jnp.swapaxes(tall_ref[...], 0, 1),
             jnp.swapaxes(talr_ref[...], 0, 1)], axis=1)


def _gather_body(rowsa_hbm, rowsb_hbm, len_hbm, table_hbm, out_hbm,
                 idxa_v, idxb_v, len_v, inv_v, bufa, bufb, outw,
                 sems):
    wid = _worker_id()

    pltpu.sync_copy(rowsa_hbm.at[wid], idxa_v)
    pltpu.sync_copy(rowsb_hbm.at[wid], idxb_v)
    pltpu.sync_copy(len_hbm.at[wid], len_v)

    for g in range(RPW // LANES):
        lens16 = len_v[pl.ds(g * LANES, LANES)]
        inv_v[pl.ds(g * LANES, LANES)] = (
            1.0 / jnp.maximum(lens16, 1).astype(jnp.float32))

    lane = lax.broadcasted_iota(jnp.int32, (LANES,), 0)
    zero = lane * 0
    sem0, sem1 = sems

    def issue(r, slot, sem):
        pltpu.async_copy(table_hbm.at[idxa_v.at[r]], bufa.at[slot], sem)
        pltpu.async_copy(table_hbm.at[idxb_v.at[r]], bufb.at[slot], sem)

    def drain(slot, sem):
        pltpu.make_async_copy(table_hbm.at[pl.ds(0, CA)],
                              bufa.at[slot], sem).wait()
        pltpu.make_async_copy(table_hbm.at[pl.ds(0, CB)],
                              bufb.at[slot], sem).wait()

    def accumulate(r, slot):
        def make_acc(buf):
            def acc_body(i, accs):
                t = i * 2
                new = []
                for c in range(NCHUNK):
                    new.append(accs[c] + buf[slot, t, pl.ds(c * LANES, LANES)])
                for c in range(NCHUNK):
                    new.append(accs[NCHUNK + c]
                               + buf[slot, t + 1, pl.ds(c * LANES, LANES)])
                return tuple(new)
            return acc_body

        accs = tuple(jnp.zeros((LANES,), jnp.float32)
                     for _ in range(2 * NCHUNK))
        accs = lax.fori_loop(0, CA // 2, make_acc(bufa), accs, unroll=4)
        accs = lax.fori_loop(0, CB // 2, make_acc(bufb), accs, unroll=4)

        sinv = plsc.load_gather(inv_v, [zero + r])
        for c in range(NCHUNK):
            outw[r, pl.ds(c * LANES, LANES)] = (
                (accs[c] + accs[NCHUNK + c]) * sinv)

    issue(0, 0, sem0)

    def pair_body(p, carry):
        r0 = 2 * p
        r1 = r0 + 1
        issue(r1, 1, sem1)
        drain(0, sem0)
        accumulate(r0, 0)
        issue(jnp.minimum(r1 + 1, RPW - 1), 0, sem0)
        drain(1, sem1)
        accumulate(r1, 1)
        return carry

    lax.fori_loop(0, RPW // 2, pair_body, 0)
    drain(0, sem0)
    pltpu.sync_copy(outw, out_hbm.at[pl.ds(wid * RPW, RPW)])


@functools.partial(jax.jit, static_argnames=("interpret",))
def _run(tokens, token_lengths, table, interpret=False):
    mesh = plsc.VectorSubcoreMesh(core_axis_name="c", subcore_axis_name="s",
                                  num_cores=2, num_subcores=16)
    tabt = table.T
    tall = tabt[:, VFULL:VFULL + TAILN]
    talr = tabt[:, VFULL + TAILN:]
    pairs_tab = pl.pallas_call(
        _tr_body,
        grid=(NSTEP,),
        in_specs=[
            pl.BlockSpec((D, TW), lambda i: (0, jnp.minimum(2 * i,
                                                            NBLK - 2))),
            pl.BlockSpec((D, TW), lambda i: (0, jnp.minimum(2 * i + 1,
                                                            NBLK - 1))),
            pl.BlockSpec((D, TAILN), lambda i: (0, 0)),
            pl.BlockSpec((D, TAILN), lambda i: (0, 0)),
        ],
        out_specs=pl.BlockSpec((TW, 2 * D), lambda i: (i, 0)),
        out_shape=jax.ShapeDtypeStruct((NPAIR, 2 * D), jnp.float32),
        interpret=interpret,
    )(tabt, tabt, tall, talr)

    tab_lin = pairs_tab.reshape(2 * NPAIR, D)

    blk = tokens >> 11
    half = blk & 1
    prow = (blk >> 1) * TW + (tokens & (TW - 1))
    tail_j = tokens - VFULL
    is_tail = tokens >= VFULL
    half = jnp.where(is_tail, tail_j // TAILN, half)
    prow = jnp.where(is_tail, (NSTEP - 1) * TW + tail_j % TAILN, prow)
    rows = 2 * prow + half
    rowsa = rows[:, :CA].reshape(NUM_WORKERS, RPW, CA)
    rowsb = rows[:, CA:].reshape(NUM_WORKERS, RPW, CB)
    lens = token_lengths.reshape(NUM_WORKERS, RPW)
    gather_k = pl.kernel(
        _gather_body,
        out_type=jax.ShapeDtypeStruct((B, D), jnp.float32),
        mesh=mesh,
        compiler_params=pltpu.CompilerParams(needs_layout_passes=False,
                                             use_tc_tiling_on_sc=False),
        scratch_types=[
            pltpu.VMEM((RPW, CA), jnp.int32),
            pltpu.VMEM((RPW, CB), jnp.int32),
            pltpu.VMEM((RPW,), jnp.int32),
            pltpu.VMEM((RPW,), jnp.float32),
            pltpu.VMEM((2, CA, D), jnp.float32),
            pltpu.VMEM((2, CB, D), jnp.float32),
            pltpu.VMEM((RPW, D), jnp.float32),
            (pltpu.SemaphoreType.DMA, pltpu.SemaphoreType.DMA),
        ],
        interpret=interpret,
    )
    return gather_k(rowsa, rowsb, lens, tab_lin)


def kernel(tokens, token_lengths, table):
    return _run(tokens, token_lengths, table)


# speedup vs baseline: 4.6648x; 1.2789x over previous
import functools

import jax
import jax.numpy as jnp
from jax import lax
from jax.experimental import pallas as pl
from jax.experimental.pallas import tpu as pltpu
from jax.experimental.pallas import tpu_sc as plsc

B = 4096
L = 200
D = 64
NUM_WORKERS = 32
RPW = B // NUM_WORKERS
CA = 104
CB = L - CA
LANES = 16
NCHUNK = D // LANES
V = 1000000
TW = 8192
TWLOG = 13
NBLK = V // TW
NSTEP = NBLK // 2 + 1
VFULL = NBLK * TW
TAILN = (V - VFULL) // 2
NPAIR = NSTEP * TW


def _worker_id():
    return lax.axis_index("s") * 2 + lax.axis_index("c")


def _tr_body(srcl_ref, srcr_ref, tall_ref, talr_ref, dst_ref):
    i = pl.program_id(0)

    @pl.when(i < NSTEP - 1)
    def _():
        dst_ref[...] = jnp.concatenate(
            [jnp.swapaxes(srcl_ref[...], 0, 1),
             jnp.swapaxes(srcr_ref[...], 0, 1)], axis=1)

    @pl.when(i == NSTEP - 1)
    def _():
        dst_ref[pl.ds(0, TAILN), :] = jnp.concatenate(
            [jnp.swapaxes(tall_ref[...], 0, 1),
             jnp.swapaxes(talr_ref[...], 0, 1)], axis=1)


def _gather_body(rowsa_hbm, rowsb_hbm, len_hbm, table_hbm, out_hbm,
                 idxa_v, idxb_v, len_v, inv_v, bufa, bufb, outw,
                 sems):
    wid = _worker_id()

    pltpu.sync_copy(rowsa_hbm.at[wid], idxa_v)
    pltpu.sync_copy(rowsb_hbm.at[wid], idxb_v)
    pltpu.sync_copy(len_hbm.at[wid], len_v)

    for g in range(RPW // LANES):
        lens16 = len_v[pl.ds(g * LANES, LANES)]
        inv_v[pl.ds(g * LANES, LANES)] = (
            1.0 / jnp.maximum(lens16, 1).astype(jnp.float32))

    lane = lax.broadcasted_iota(jnp.int32, (LANES,), 0)
    zero = lane * 0
    sem0, sem1 = sems

    def issue(r, slot, sem):
        pltpu.async_copy(table_hbm.at[idxa_v.at[r]], bufa.at[slot], sem)
        pltpu.async_copy(table_hbm.at[idxb_v.at[r]], bufb.at[slot], sem)

    def drain(slot, sem):
        pltpu.make_async_copy(table_hbm.at[pl.ds(0, CA)],
                              bufa.at[slot], sem).wait()
        pltpu.make_async_copy(table_hbm.at[pl.ds(0, CB)],
                              bufb.at[slot], sem).wait()

    def accumulate(r, slot):
        def make_acc(buf):
            def acc_body(i, accs):
                t = i * 2
                new = []
                for c in range(NCHUNK):
                    new.append(accs[c] + buf[slot, t, pl.ds(c * LANES, LANES)])
                for c in range(NCHUNK):
                    new.append(accs[NCHUNK + c]
                               + buf[slot, t + 1, pl.ds(c * LANES, LANES)])
                return tuple(new)
            return acc_body

        accs = tuple(jnp.zeros((LANES,), jnp.float32)
                     for _ in range(2 * NCHUNK))
        accs = lax.fori_loop(0, CA // 2, make_acc(bufa), accs, unroll=4)
        accs = lax.fori_loop(0, CB // 2, make_acc(bufb), accs, unroll=4)

        sinv = plsc.load_gather(inv_v, [zero + r])
        for c in range(NCHUNK):
            outw[r, pl.ds(c * LANES, LANES)] = (
                (accs[c] + accs[NCHUNK + c]) * sinv)

    issue(0, 0, sem0)

    def pair_body(p, carry):
        r0 = 2 * p
        r1 = r0 + 1
        issue(r1, 1, sem1)
        drain(0, sem0)
        accumulate(r0, 0)
        issue(jnp.minimum(r1 + 1, RPW - 1), 0, sem0)
        drain(1, sem1)
        accumulate(r1, 1)
        return carry

    lax.fori_loop(0, RPW // 2, pair_body, 0)
    drain(0, sem0)
    pltpu.sync_copy(outw, out_hbm.at[pl.ds(wid * RPW, RPW)])


@functools.partial(jax.jit, static_argnames=("interpret",))
def _run(tokens, token_lengths, table, interpret=False):
    mesh = plsc.VectorSubcoreMesh(core_axis_name="c", subcore_axis_name="s",
                                  num_cores=2, num_subcores=16)
    tabt = table.T
    tall = tabt[:, VFULL:VFULL + TAILN]
    talr = tabt[:, VFULL + TAILN:]
    pairs_tab = pl.pallas_call(
        _tr_body,
        grid=(NSTEP,),
        in_specs=[
            pl.BlockSpec((D, TW), lambda i: (0, jnp.minimum(2 * i,
                                                            NBLK - 2))),
            pl.BlockSpec((D, TW), lambda i: (0, jnp.minimum(2 * i + 1,
                                                            NBLK - 1))),
            pl.BlockSpec((D, TAILN), lambda i: (0, 0)),
            pl.BlockSpec((D, TAILN), lambda i: (0, 0)),
        ],
        out_specs=pl.BlockSpec((TW, 2 * D), lambda i: (i, 0)),
        out_shape=jax.ShapeDtypeStruct((NPAIR, 2 * D), jnp.float32),
        interpret=interpret,
    )(tabt, tabt, tall, talr)

    tab_lin = pairs_tab.reshape(2 * NPAIR, D)

    blk = tokens >> TWLOG
    half = blk & 1
    prow = (blk >> 1) * TW + (tokens & (TW - 1))
    tail_j = tokens - VFULL
    is_tail = tokens >= VFULL
    half = jnp.where(is_tail, tail_j // TAILN, half)
    prow = jnp.where(is_tail, (NSTEP - 1) * TW + tail_j % TAILN, prow)
    rows = 2 * prow + half
    rowsa = rows[:, :CA].reshape(NUM_WORKERS, RPW, CA)
    rowsb = rows[:, CA:].reshape(NUM_WORKERS, RPW, CB)
    lens = token_lengths.reshape(NUM_WORKERS, RPW)
    gather_k = pl.kernel(
        _gather_body,
        out_type=jax.ShapeDtypeStruct((B, D), jnp.float32),
        mesh=mesh,
        compiler_params=pltpu.CompilerParams(needs_layout_passes=False,
                                             use_tc_tiling_on_sc=False),
        scratch_types=[
            pltpu.VMEM((RPW, CA), jnp.int32),
            pltpu.VMEM((RPW, CB), jnp.int32),
            pltpu.VMEM((RPW,), jnp.int32),
            pltpu.VMEM((RPW,), jnp.float32),
            pltpu.VMEM((2, CA, D), jnp.float32),
            pltpu.VMEM((2, CB, D), jnp.float32),
            pltpu.VMEM((RPW, D), jnp.float32),
            (pltpu.SemaphoreType.DMA, pltpu.SemaphoreType.DMA),
        ],
        interpret=interpret,
    )
    return gather_k(rowsa, rowsb, lens, tab_lin)


def kernel(tokens, token_lengths, table):
    return _run(tokens, token_lengths, table)


# speedup vs baseline: 4.7745x; 1.0235x over previous
import functools

import jax
import jax.numpy as jnp
from jax import lax
from jax.experimental import pallas as pl
from jax.experimental.pallas import tpu as pltpu
from jax.experimental.pallas import tpu_sc as plsc

B = 4096
L = 200
D = 64
NUM_WORKERS = 32
RPW = B // NUM_WORKERS
CA = 104
CB = L - CA
LANES = 16
NCHUNK = D // LANES
V = 1000000
TW = 16384
TWLOG = 14
NBLK = V // TW
NSTEP = (NBLK + 1) // 2 + 1
VFULL = NBLK * TW
TAILN = (V - VFULL) // 2
NPAIR = NSTEP * TW


def _worker_id():
    return lax.axis_index("s") * 2 + lax.axis_index("c")


def _tr_body(srcl_ref, srcr_ref, tall_ref, talr_ref, dst_ref):
    i = pl.program_id(0)

    @pl.when(i < NSTEP - 1)
    def _():
        dst_ref[...] = jnp.concatenate(
            [jnp.swapaxes(srcl_ref[...], 0, 1),
             jnp.swapaxes(srcr_ref[...], 0, 1)], axis=1)

    @pl.when(i == NSTEP - 1)
    def _():
        dst_ref[pl.ds(0, TAILN), :] = jnp.concatenate(
            [jnp.swapaxes(tall_ref[...], 0, 1),
             jnp.swapaxes(talr_ref[...], 0, 1)], axis=1)


def _gather_body(rowsa_hbm, rowsb_hbm, len_hbm, table_hbm, out_hbm,
                 idxa_v, idxb_v, len_v, inv_v, bufa, bufb, outw,
                 sems):
    wid = _worker_id()

    pltpu.sync_copy(rowsa_hbm.at[wid], idxa_v)
    pltpu.sync_copy(rowsb_hbm.at[wid], idxb_v)
    pltpu.sync_copy(len_hbm.at[wid], len_v)

    for g in range(RPW // LANES):
        lens16 = len_v[pl.ds(g * LANES, LANES)]
        inv_v[pl.ds(g * LANES, LANES)] = (
            1.0 / jnp.maximum(lens16, 1).astype(jnp.float32))

    lane = lax.broadcasted_iota(jnp.int32, (LANES,), 0)
    zero = lane * 0
    sem0, sem1 = sems

    def issue(r, slot, sem):
        pltpu.async_copy(table_hbm.at[idxa_v.at[r]], bufa.at[slot], sem)
        pltpu.async_copy(table_hbm.at[idxb_v.at[r]], bufb.at[slot], sem)

    def drain(slot, sem):
        pltpu.make_async_copy(table_hbm.at[pl.ds(0, CA)],
                              bufa.at[slot], sem).wait()
        pltpu.make_async_copy(table_hbm.at[pl.ds(0, CB)],
                              bufb.at[slot], sem).wait()

    def accumulate(r, slot):
        def make_acc(buf):
            def acc_body(i, accs):
                t = i * 2
                new = []
                for c in range(NCHUNK):
                    new.append(accs[c] + buf[slot, t, pl.ds(c * LANES, LANES)])
                for c in range(NCHUNK):
                    new.append(accs[NCHUNK + c]
                               + buf[slot, t + 1, pl.ds(c * LANES, LANES)])
                return tuple(new)
            return acc_body

        accs = tuple(jnp.zeros((LANES,), jnp.float32)
                     for _ in range(2 * NCHUNK))
        accs = lax.fori_loop(0, CA // 2, make_acc(bufa), accs, unroll=4)
        accs = lax.fori_loop(0, CB // 2, make_acc(bufb), accs, unroll=4)

        sinv = plsc.load_gather(inv_v, [zero + r])
        for c in range(NCHUNK):
            outw[r, pl.ds(c * LANES, LANES)] = (
                (accs[c] + accs[NCHUNK + c]) * sinv)

    issue(0, 0, sem0)

    def pair_body(p, carry):
        r0 = 2 * p
        r1 = r0 + 1
        issue(r1, 1, sem1)
        drain(0, sem0)
        accumulate(r0, 0)
        issue(jnp.minimum(r1 + 1, RPW - 1), 0, sem0)
        drain(1, sem1)
        accumulate(r1, 1)
        return carry

    lax.fori_loop(0, RPW // 2, pair_body, 0)
    drain(0, sem0)
    pltpu.sync_copy(outw, out_hbm.at[pl.ds(wid * RPW, RPW)])


@functools.partial(jax.jit, static_argnames=("interpret",))
def _run(tokens, token_lengths, table, interpret=False):
    mesh = plsc.VectorSubcoreMesh(core_axis_name="c", subcore_axis_name="s",
                                  num_cores=2, num_subcores=16)
    tabt = table.T
    tall = tabt[:, VFULL:VFULL + TAILN]
    talr = tabt[:, VFULL + TAILN:]
    pairs_tab = pl.pallas_call(
        _tr_body,
        grid=(NSTEP,),
        in_specs=[
            pl.BlockSpec((D, TW), lambda i: (0, jnp.minimum(2 * i,
                                                            NBLK - 1))),
            pl.BlockSpec((D, TW), lambda i: (0, jnp.minimum(2 * i + 1,
                                                            NBLK - 1))),
            pl.BlockSpec((D, TAILN), lambda i: (0, 0)),
            pl.BlockSpec((D, TAILN), lambda i: (0, 0)),
        ],
        out_specs=pl.BlockSpec((TW, 2 * D), lambda i: (i, 0)),
        out_shape=jax.ShapeDtypeStruct((NPAIR, 2 * D), jnp.float32),
        interpret=interpret,
    )(tabt, tabt, tall, talr)

    tab_lin = pairs_tab.reshape(2 * NPAIR, D)

    blk = tokens >> TWLOG
    half = blk & 1
    prow = (blk >> 1) * TW + (tokens & (TW - 1))
    tail_j = tokens - VFULL
    is_tail = tokens >= VFULL
    half = jnp.where(is_tail, tail_j // TAILN, half)
    prow = jnp.where(is_tail, (NSTEP - 1) * TW + tail_j % TAILN, prow)
    rows = 2 * prow + half
    rowsa = rows[:, :CA].reshape(NUM_WORKERS, RPW, CA)
    rowsb = rows[:, CA:].reshape(NUM_WORKERS, RPW, CB)
    lens = token_lengths.reshape(NUM_WORKERS, RPW)
    gather_k = pl.kernel(
        _gather_body,
        out_type=jax.ShapeDtypeStruct((B, D), jnp.float32),
        mesh=mesh,
        compiler_params=pltpu.CompilerParams(needs_layout_passes=False,
                                             use_tc_tiling_on_sc=False),
        scratch_types=[
            pltpu.VMEM((RPW, CA), jnp.int32),
            pltpu.VMEM((RPW, CB), jnp.int32),
            pltpu.VMEM((RPW,), jnp.int32),
            pltpu.VMEM((RPW,), jnp.float32),
            pltpu.VMEM((2, CA, D), jnp.float32),
            pltpu.VMEM((2, CB, D), jnp.float32),
            pltpu.VMEM((RPW, D), jnp.float32),
            (pltpu.SemaphoreType.DMA, pltpu.SemaphoreType.DMA),
        ],
        interpret=interpret,
    )
    return gather_k(rowsa, rowsb, lens, tab_lin)


def kernel(tokens, token_lengths, table):
    return _run(tokens, token_lengths, table)


# speedup vs baseline: 4.7829x; 1.0018x over previous
import jax
import jax.numpy as jnp
from jax import lax
from jax.experimental import pallas as pl
from jax.experimental.pallas import tpu as pltpu
from jax.experimental.pallas import tpu_sc as plsc

B = 4096
L = 200
D = 64
NUM_WORKERS = 32
RPW = B // NUM_WORKERS
CA = 104
CB = L - CA
LANES = 16
NCHUNK = D // LANES
V = 1000000
TW = 16384
TWLOG = 14
NBLK = V // TW
NSTEP = (NBLK + 1) // 2 + 1
VFULL = NBLK * TW
TAILN = (V - VFULL) // 2
NPAIR = NSTEP * TW


def _worker_id():
    return lax.axis_index("s") * 2 + lax.axis_index("c")


def _tr_body(srcl_ref, srcr_ref, tall_ref, talr_ref, dst_ref):
    i = pl.program_id(0)

    @pl.when(i < NSTEP - 1)
    def _():
        dst_ref[...] = jnp.concatenate(
            [jnp.swapaxes(srcl_ref[...], 0, 1),
             jnp.swapaxes(srcr_ref[...], 0, 1)], axis=1)

    @pl.when(i == NSTEP - 1)
    def _():
        dst_ref[pl.ds(0, TAILN), :] = jnp.concatenate(
            [jnp.swapaxes(tall_ref[...], 0, 1),
             jnp.swapaxes(talr_ref[...], 0, 1)], axis=1)


def _gather_body(rowsa_hbm, rowsb_hbm, len_hbm, table_hbm, out_hbm,
                 idxa_v, idxb_v, len_v, inv_v, bufa, bufb, outw,
                 sems):
    wid = _worker_id()

    pltpu.sync_copy(rowsa_hbm.at[wid], idxa_v)
    pltpu.sync_copy(rowsb_hbm.at[wid], idxb_v)
    pltpu.sync_copy(len_hbm.at[wid], len_v)

    for g in range(RPW // LANES):
        lens16 = len_v[pl.ds(g * LANES, LANES)]
        inv_v[pl.ds(g * LANES, LANES)] = (
            1.0 / jnp.maximum(lens16, 1).astype(jnp.float32))

    lane = lax.broadcasted_iota(jnp.int32, (LANES,), 0)
    zero = lane * 0
    sem0, sem1 = sems

    def issue(r, slot, sem):
        pltpu.async_copy(table_hbm.at[idxa_v.at[r]], bufa.at[slot], sem)
        pltpu.async_copy(table_hbm.at[idxb_v.at[r]], bufb.at[slot], sem)

    def drain(slot, sem):
        pltpu.make_async_copy(table_hbm.at[pl.ds(0, CA)],
                              bufa.at[slot], sem).wait()
        pltpu.make_async_copy(table_hbm.at[pl.ds(0, CB)],
                              bufb.at[slot], sem).wait()

    def accumulate(r, slot):
        def make_acc(buf):
            def acc_body(i, accs):
                t = i * 2
                new = []
                for c in range(NCHUNK):
                    new.append(accs[c] + buf[slot, t, pl.ds(c * LANES, LANES)])
                for c in range(NCHUNK):
                    new.append(accs[NCHUNK + c]
                               + buf[slot, t + 1, pl.ds(c * LANES, LANES)])
                return tuple(new)
            return acc_body

        accs = tuple(jnp.zeros((LANES,), jnp.float32)
                     for _ in range(2 * NCHUNK))
        accs = lax.fori_loop(0, CA // 2, make_acc(bufa), accs, unroll=4)
        accs = lax.fori_loop(0, CB // 2, make_acc(bufb), accs, unroll=4)

        sinv = plsc.load_gather(inv_v, [zero + r])
        for c in range(NCHUNK):
            outw[r, pl.ds(c * LANES, LANES)] = (
                (accs[c] + accs[NCHUNK + c]) * sinv)

    issue(0, 0, sem0)

    def pair_body(p, carry):
        r0 = 2 * p
        r1 = r0 + 1
        issue(r1, 1, sem1)
        drain(0, sem0)
        accumulate(r0, 0)
        issue(jnp.minimum(r1 + 1, RPW - 1), 0, sem0)
        drain(1, sem1)
        accumulate(r1, 1)
        return carry

    lax.fori_loop(0, RPW // 2, pair_body, 0)
    drain(0, sem0)
    pltpu.sync_copy(outw, out_hbm.at[pl.ds(wid * RPW, RPW)])


@jax.jit
def _run(tokens, token_lengths, table):
    mesh = plsc.VectorSubcoreMesh(core_axis_name="c", subcore_axis_name="s",
                                  num_cores=2, num_subcores=16)
    tabt = table.T
    tall = tabt[:, VFULL:VFULL + TAILN]
    talr = tabt[:, VFULL + TAILN:]
    pairs_tab = pl.pallas_call(
        _tr_body,
        grid=(NSTEP,),
        in_specs=[
            pl.BlockSpec((D, TW), lambda i: (0, jnp.minimum(2 * i,
                                                            NBLK - 1))),
            pl.BlockSpec((D, TW), lambda i: (0, jnp.minimum(2 * i + 1,
                                                            NBLK - 1))),
            pl.BlockSpec((D, TAILN), lambda i: (0, 0)),
            pl.BlockSpec((D, TAILN), lambda i: (0, 0)),
        ],
        out_specs=pl.BlockSpec((TW, 2 * D), lambda i: (i, 0)),
        out_shape=jax.ShapeDtypeStruct((NPAIR, 2 * D), jnp.float32),
    )(tabt, tabt, tall, talr)

    tab_lin = pairs_tab.reshape(2 * NPAIR, D)

    blk = tokens >> TWLOG
    half = blk & 1
    prow = (blk >> 1) * TW + (tokens & (TW - 1))
    tail_j = tokens - VFULL
    is_tail = tokens >= VFULL
    half = jnp.where(is_tail, tail_j // TAILN, half)
    prow = jnp.where(is_tail, (NSTEP - 1) * TW + tail_j % TAILN, prow)
    rows = 2 * prow + half
    rowsa = rows[:, :CA].reshape(NUM_WORKERS, RPW, CA)
    rowsb = rows[:, CA:].reshape(NUM_WORKERS, RPW, CB)
    lens = token_lengths.reshape(NUM_WORKERS, RPW)
    gather_k = pl.kernel(
        _gather_body,
        out_type=jax.ShapeDtypeStruct((B, D), jnp.float32),
        mesh=mesh,
        compiler_params=pltpu.CompilerParams(needs_layout_passes=False,
                                             use_tc_tiling_on_sc=False),
        scratch_types=[
            pltpu.VMEM((RPW, CA), jnp.int32),
            pltpu.VMEM((RPW, CB), jnp.int32),
            pltpu.VMEM((RPW,), jnp.int32),
            pltpu.VMEM((RPW,), jnp.float32),
            pltpu.VMEM((2, CA, D), jnp.float32),
            pltpu.VMEM((2, CB, D), jnp.float32),
            pltpu.VMEM((RPW, D), jnp.float32),
            (pltpu.SemaphoreType.DMA, pltpu.SemaphoreType.DMA),
        ],
    )
    return gather_k(rowsa, rowsb, lens, tab_lin)


def kernel(tokens, token_lengths, table):
    return _run(tokens, token_lengths, table)
